# baseline (device time: 43771 ns/iter reference)
import jax
import jax.numpy as jnp
from jax import lax
from jax.experimental import pallas as pl
from jax.experimental.pallas import tpu as pltpu

N_DEV = 8


def kernel(x, w_mat):
    m_per, k = x.shape
    _, n_per = w_mat.shape

    def body(x_ref, w_ref, out_ref, xfull_ref, send_sems, recv_sems):
        my = lax.axis_index("i")

        xfull_ref[pl.ds(my * m_per, m_per), :] = x_ref[...]

        sends = []
        for off in range(1, N_DEV):
            tgt = lax.rem(my + off, N_DEV)
            rdma = pltpu.make_async_remote_copy(
                src_ref=x_ref,
                dst_ref=xfull_ref.at[pl.ds(my * m_per, m_per), :],
                send_sem=send_sems.at[off],
                recv_sem=recv_sems.at[off],
                device_id=(tgt,),
                device_id_type=pl.DeviceIdType.MESH,
            )
            rdma.start()
            sends.append(rdma)

        for off in range(1, N_DEV):
            origin = lax.rem(my - off + N_DEV, N_DEV)
            recv = pltpu.make_async_remote_copy(
                src_ref=x_ref,
                dst_ref=xfull_ref.at[pl.ds(origin * m_per, m_per), :],
                send_sem=send_sems.at[0],
                recv_sem=recv_sems.at[off],
                device_id=(my,),
                device_id_type=pl.DeviceIdType.MESH,
            )
            recv.wait_recv()

        acc = jnp.dot(
            xfull_ref[...], w_ref[...], preferred_element_type=jnp.float32
        )
        out_ref[...] = jnp.maximum(acc, 0.0)

        for rdma in sends:
            rdma.wait_send()

    return pl.pallas_call(
        body,
        out_shape=jax.ShapeDtypeStruct((N_DEV * m_per, n_per), jnp.float32),
        in_specs=[
            pl.BlockSpec(memory_space=pltpu.VMEM),
            pl.BlockSpec(memory_space=pltpu.VMEM),
        ],
        out_specs=pl.BlockSpec(memory_space=pltpu.VMEM),
        scratch_shapes=[
            pltpu.VMEM((N_DEV * m_per, k), x.dtype),
            pltpu.SemaphoreType.DMA((N_DEV,)),
            pltpu.SemaphoreType.DMA((N_DEV,)),
        ],
    )(x, w_mat)


# device time: 38458 ns/iter; 1.1382x vs baseline; 1.1382x over previous
import jax
import jax.numpy as jnp
from jax import lax
from jax.experimental import pallas as pl
from jax.experimental.pallas import tpu as pltpu

N_DEV = 8


def kernel(x, w_mat):
    m_per, k = x.shape
    _, n_per = w_mat.shape

    def body(x_ref, w_ref, out_ref, xfull_ref, send_sems, recv_sems):
        my = lax.axis_index("i")

        barrier_sem = pltpu.get_barrier_semaphore()
        for off in range(1, N_DEV):
            pl.semaphore_signal(
                barrier_sem,
                inc=1,
                device_id=(lax.rem(my + off, N_DEV),),
                device_id_type=pl.DeviceIdType.MESH,
            )
        pl.semaphore_wait(barrier_sem, N_DEV - 1)

        with jax.named_scope("copy_own"):
            xfull_ref[pl.ds(my * m_per, m_per), :] = x_ref[...]

        sends = []
        with jax.named_scope("issue_sends"):
            for off in range(1, N_DEV):
                tgt = lax.rem(my + off, N_DEV)
                rdma = pltpu.make_async_remote_copy(
                    src_ref=x_ref,
                    dst_ref=xfull_ref.at[pl.ds(my * m_per, m_per), :],
                    send_sem=send_sems.at[off],
                    recv_sem=recv_sems.at[off],
                    device_id=(tgt,),
                    device_id_type=pl.DeviceIdType.MESH,
                )
                rdma.start()
                sends.append(rdma)

        for off in range(1, N_DEV):
            with jax.named_scope(f"wait_recv#off={off}"):
                origin = lax.rem(my - off + N_DEV, N_DEV)
                recv = pltpu.make_async_remote_copy(
                    src_ref=x_ref,
                    dst_ref=xfull_ref.at[pl.ds(origin * m_per, m_per), :],
                    send_sem=send_sems.at[0],
                    recv_sem=recv_sems.at[off],
                    device_id=(my,),
                    device_id_type=pl.DeviceIdType.MESH,
                )
                recv.wait_recv()

        with jax.named_scope("gemm"):
            acc = jnp.dot(
                xfull_ref[...], w_ref[...], preferred_element_type=jnp.float32
            )
            out_ref[...] = jnp.maximum(acc, 0.0)

        with jax.named_scope("wait_sends"):
            for rdma in sends:
                rdma.wait_send()

    return pl.pallas_call(
        body,
        out_shape=jax.ShapeDtypeStruct((N_DEV * m_per, n_per), jnp.float32),
        in_specs=[
            pl.BlockSpec(memory_space=pltpu.VMEM),
            pl.BlockSpec(memory_space=pltpu.VMEM),
        ],
        out_specs=pl.BlockSpec(memory_space=pltpu.VMEM),
        scratch_shapes=[
            pltpu.VMEM((N_DEV * m_per, k), x.dtype),
            pltpu.SemaphoreType.DMA((N_DEV,)),
            pltpu.SemaphoreType.DMA((N_DEV,)),
        ],
        compiler_params=pltpu.CompilerParams(collective_id=0),
    )(x, w_mat)


# device time: 24256 ns/iter; 1.8045x vs baseline; 1.5855x over previous
import jax
import jax.numpy as jnp
from jax import lax
from jax.experimental import pallas as pl
from jax.experimental.pallas import tpu as pltpu

N_DEV = 8


def kernel(x, w_mat):
    m_per, k = x.shape
    _, n_per = w_mat.shape

    def body(x_ref, w_ref, out_ref, xfull_ref, send_sems, recv_sems):
        my = lax.axis_index("i")

        barrier_sem = pltpu.get_barrier_semaphore()
        for off in range(1, N_DEV):
            pl.semaphore_signal(
                barrier_sem,
                inc=1,
                device_id=(lax.rem(my + off, N_DEV),),
                device_id_type=pl.DeviceIdType.MESH,
            )
        pl.semaphore_wait(barrier_sem, N_DEV - 1)

        with jax.named_scope("copy_own"):
            xfull_ref[pl.ds(my * m_per, m_per), :] = x_ref[...].astype(
                jnp.bfloat16
            )

        sends = []
        with jax.named_scope("issue_sends"):
            for off in range(1, N_DEV):
                tgt = lax.rem(my + off, N_DEV)
                rdma = pltpu.make_async_remote_copy(
                    src_ref=xfull_ref.at[pl.ds(my * m_per, m_per), :],
                    dst_ref=xfull_ref.at[pl.ds(my * m_per, m_per), :],
                    send_sem=send_sems.at[off],
                    recv_sem=recv_sems.at[off],
                    device_id=(tgt,),
                    device_id_type=pl.DeviceIdType.MESH,
                )
                rdma.start()
                sends.append(rdma)

        for off in range(1, N_DEV):
            with jax.named_scope(f"wait_recv#off={off}"):
                origin = lax.rem(my - off + N_DEV, N_DEV)
                recv = pltpu.make_async_remote_copy(
                    src_ref=xfull_ref.at[pl.ds(origin * m_per, m_per), :],
                    dst_ref=xfull_ref.at[pl.ds(origin * m_per, m_per), :],
                    send_sem=send_sems.at[0],
                    recv_sem=recv_sems.at[off],
                    device_id=(my,),
                    device_id_type=pl.DeviceIdType.MESH,
                )
                recv.wait_recv()

        with jax.named_scope("gemm"):
            acc = jnp.dot(
                xfull_ref[...],
                w_ref[...].astype(jnp.bfloat16),
                preferred_element_type=jnp.float32,
            )
            out_ref[...] = jnp.maximum(acc, 0.0)

        with jax.named_scope("wait_sends"):
            for rdma in sends:
                rdma.wait_send()

    return pl.pallas_call(
        body,
        out_shape=jax.ShapeDtypeStruct((N_DEV * m_per, n_per), jnp.float32),
        in_specs=[
            pl.BlockSpec(memory_space=pltpu.VMEM),
            pl.BlockSpec(memory_space=pltpu.VMEM),
        ],
        out_specs=pl.BlockSpec(memory_space=pltpu.VMEM),
        scratch_shapes=[
            pltpu.VMEM((N_DEV * m_per, k), jnp.bfloat16),
            pltpu.SemaphoreType.DMA((N_DEV,)),
            pltpu.SemaphoreType.DMA((N_DEV,)),
        ],
        compiler_params=pltpu.CompilerParams(collective_id=0),
    )(x, w_mat)


# device time: 23683 ns/iter; 1.8482x vs baseline; 1.0242x over previous
import jax
import jax.numpy as jnp
from jax import lax
from jax.experimental import pallas as pl
from jax.experimental.pallas import tpu as pltpu

N_DEV = 8


def kernel(x, w_mat):
    m_per, k = x.shape
    _, n_per = w_mat.shape

    def body(x_ref, w_ref, out_ref, xfull_ref, wb_ref, send_sems, recv_sems):
        my = lax.axis_index("i")

        barrier_sem = pltpu.get_barrier_semaphore()
        for off in range(1, N_DEV):
            pl.semaphore_signal(
                barrier_sem,
                inc=1,
                device_id=(lax.rem(my + off, N_DEV),),
                device_id_type=pl.DeviceIdType.MESH,
            )
        pl.semaphore_wait(barrier_sem, N_DEV - 1)

        with jax.named_scope("copy_own"):
            xfull_ref[pl.ds(my * m_per, m_per), :] = x_ref[...].astype(
                jnp.bfloat16
            )

        sends = []
        with jax.named_scope("issue_sends"):
            for off in range(1, N_DEV):
                tgt = lax.rem(my + off, N_DEV)
                rdma = pltpu.make_async_remote_copy(
                    src_ref=xfull_ref.at[pl.ds(my * m_per, m_per), :],
                    dst_ref=xfull_ref.at[pl.ds(my * m_per, m_per), :],
                    send_sem=send_sems.at[off],
                    recv_sem=recv_sems.at[off],
                    device_id=(tgt,),
                    device_id_type=pl.DeviceIdType.MESH,
                )
                rdma.start()
                sends.append(rdma)

        with jax.named_scope("convert_w"):
            wb_ref[...] = w_ref[...].astype(jnp.bfloat16)

        def block_gemm(origin):
            chunk = xfull_ref[pl.ds(origin * m_per, m_per), :]
            acc = jnp.dot(
                chunk, wb_ref[...], preferred_element_type=jnp.float32
            )
            out_ref[pl.ds(origin * m_per, m_per), :] = jnp.maximum(acc, 0.0)

        with jax.named_scope("gemm_own"):
            block_gemm(my)

        for off in range(1, N_DEV):
            with jax.named_scope(f"wait_recv#off={off}"):
                origin = lax.rem(my - off + N_DEV, N_DEV)
                recv = pltpu.make_async_remote_copy(
                    src_ref=xfull_ref.at[pl.ds(origin * m_per, m_per), :],
                    dst_ref=xfull_ref.at[pl.ds(origin * m_per, m_per), :],
                    send_sem=send_sems.at[0],
                    recv_sem=recv_sems.at[off],
                    device_id=(my,),
                    device_id_type=pl.DeviceIdType.MESH,
                )
                recv.wait_recv()
            with jax.named_scope(f"gemm#off={off}"):
                block_gemm(origin)

        with jax.named_scope("wait_sends"):
            for rdma in sends:
                rdma.wait_send()

    return pl.pallas_call(
        body,
        out_shape=jax.ShapeDtypeStruct((N_DEV * m_per, n_per), jnp.float32),
        in_specs=[
            pl.BlockSpec(memory_space=pltpu.VMEM),
            pl.BlockSpec(memory_space=pltpu.VMEM),
        ],
        out_specs=pl.BlockSpec(memory_space=pltpu.VMEM),
        scratch_shapes=[
            pltpu.VMEM((N_DEV * m_per, k), jnp.bfloat16),
            pltpu.VMEM((k, n_per), jnp.bfloat16),
            pltpu.SemaphoreType.DMA((N_DEV,)),
            pltpu.SemaphoreType.DMA((N_DEV,)),
        ],
        compiler_params=pltpu.CompilerParams(collective_id=0),
    )(x, w_mat)
